# Initial kernel scaffold; baseline (speedup 1.0000x reference)
import jax
import jax.numpy as jnp
from jax.experimental import pallas as pl


def kernel(x, edge_index, edge_w, W0, W1, W2, asrc0, asrc1, asrc2, adst0, adst1, adst2, we0, we1, we2, Wcls):
    raise NotImplementedError("write your pallas kernel here")


# speedup vs baseline: 2.5341x; 2.5341x over previous
import functools

import jax
import jax.numpy as jnp
from jax.experimental import pallas as pl
from jax.experimental.pallas import tpu as pltpu

NP = 4000
P = 50
E = 200
DF = 64
DH = 64
DW = 16
OUT = 16
B = 80
G = NP // B


def _gat_body(xt, ewt, w0, w1, w2, am0, am1, am2, wem,
              srcep, dstep, dstpe, mbias, wcls, out, acc):
    i = pl.program_id(0)

    @pl.when(i == 0)
    def _():
        acc[...] = jnp.zeros_like(acc)

    srcm = srcep[...]
    dstm = dstep[...]
    dpe = dstpe[...]
    mb = mbias[...]

    ewp = jax.lax.dot_general(wem[...], ewt[...], (((1,), (0,)), ((), ())),
                              preferred_element_type=jnp.float32)

    h = xt[...]
    for l, (wr, ar) in enumerate(((w0, am0), (w1, am1), (w2, am2))):
        z = jax.lax.dot_general(h, wr[...], (((2,), (0,)), ((), ())),
                                preferred_element_type=jnp.float32)
        uv = jax.lax.dot_general(z, ar[...], (((2,), (0,)), ((), ())),
                                 preferred_element_type=jnp.float32)
        u = uv[:, :, 0]
        v = uv[:, :, 1]
        ue = jnp.dot(srcm, u, preferred_element_type=jnp.float32)
        ve = jnp.dot(dstm, v, preferred_element_type=jnp.float32)
        e = ue + ve + ewp[l].T
        e = jnp.where(e > 0, e, 0.2 * e)
        em = mb[:, :, None] + e[None, :, :]
        m = jnp.max(em, axis=1)
        me = jnp.dot(dstm, m, preferred_element_type=jnp.float32)
        ex = jnp.exp(e - me)
        s = jnp.dot(dpe, ex, preferred_element_type=jnp.float32)
        se = jnp.dot(dstm, s, preferred_element_type=jnp.float32)
        alpha = ex / (se + 1e-9)
        zsrc = jax.lax.dot_general(srcm, z, (((1,), (0,)), ((), ())),
                                   preferred_element_type=jnp.float32)
        w3 = zsrc * alpha[:, :, None]
        agg = jax.lax.dot_general(dpe, w3, (((1,), (0,)), ((), ())),
                                  preferred_element_type=jnp.float32)
        h = jnp.where(agg > 0, agg, jnp.exp(agg) - 1.0) + h
        acc[l, :] += jnp.sum(h, axis=(0, 1))

    @pl.when(i == pl.num_programs(0) - 1)
    def _():
        mesh = acc[...] * (1.0 / (NP * P))
        o = jnp.zeros((1, OUT), jnp.float32)
        for l in range(3):
            o = o + jnp.dot(mesh[l:l + 1, :], wcls[l * DH:(l + 1) * DH, :],
                            preferred_element_type=jnp.float32)
        out[...] = o


@jax.jit
def kernel(x, edge_index, edge_w, W0, W1, W2, asrc0, asrc1, asrc2,
           adst0, adst1, adst2, we0, we1, we2, Wcls):
    src = edge_index[0].astype(jnp.int32)
    dst = edge_index[1].astype(jnp.int32)
    nodes = jnp.arange(P, dtype=jnp.int32)
    srcep = (src[:, None] == nodes[None, :]).astype(jnp.float32)
    dstep = (dst[:, None] == nodes[None, :]).astype(jnp.float32)
    dstpe = dstep.T
    mbias = (dstpe - 1.0) * 1e30

    xt = jnp.transpose(x, (1, 0, 2))
    ewt = jnp.transpose(edge_w, (2, 0, 1))

    am0 = jnp.stack([asrc0, adst0], axis=1)
    am1 = jnp.stack([asrc1, adst1], axis=1)
    am2 = jnp.stack([asrc2, adst2], axis=1)
    wem = jnp.stack([we0, we1, we2], axis=0)

    full = lambda a: pl.BlockSpec(a.shape, lambda i: (0,) * a.ndim)
    grid_spec = pltpu.PrefetchScalarGridSpec(
        num_scalar_prefetch=0,
        grid=(G,),
        in_specs=[
            pl.BlockSpec((P, B, DF), lambda i: (0, i, 0)),
            pl.BlockSpec((DW, B, E), lambda i: (0, i, 0)),
            full(W0), full(W1), full(W2),
            full(am0), full(am1), full(am2), full(wem),
            full(srcep), full(dstep), full(dstpe), full(mbias), full(Wcls),
        ],
        out_specs=pl.BlockSpec((1, OUT), lambda i: (0, 0)),
        scratch_shapes=[pltpu.VMEM((3, DH), jnp.float32)],
    )
    return pl.pallas_call(
        _gat_body,
        grid_spec=grid_spec,
        out_shape=jax.ShapeDtypeStruct((1, OUT), jnp.float32),
        compiler_params=pltpu.CompilerParams(
            dimension_semantics=("arbitrary",),
        ),
    )(xt, ewt, W0, W1, W2, am0, am1, am2, wem,
      srcep, dstep, dstpe, mbias, Wcls)


# speedup vs baseline: 2.8678x; 1.1317x over previous
import jax
import jax.numpy as jnp
from jax.experimental import pallas as pl
from jax.experimental.pallas import tpu as pltpu

NP = 4000
P = 50
E = 200
DF = 64
DH = 64
DW = 16
OUT = 16
B = 80
G = NP // B


def _gat_body(xb, ewb, w0, w1, w2, am0, am1, am2, wem,
              srcpe, dstpe, dstep, mbias, wcls, out, acc):
    i = pl.program_id(0)

    @pl.when(i == 0)
    def _():
        acc[...] = jnp.zeros_like(acc)

    spe = srcpe[...]
    dpe = dstpe[...]
    dep = dstep[...]
    mb = mbias[...]

    ewp = jax.lax.dot_general(ewb[...], wem[...], (((2,), (0,)), ((), ())),
                              preferred_element_type=jnp.float32)

    h = xb[...]
    for l, (wr, ar) in enumerate(((w0, am0), (w1, am1), (w2, am2))):
        z = jax.lax.dot_general(h, wr[...], (((2,), (0,)), ((), ())),
                                preferred_element_type=jnp.float32)
        uv = jax.lax.dot_general(z, ar[...], (((2,), (0,)), ((), ())),
                                 preferred_element_type=jnp.float32)
        u = uv[:, :, 0]
        v = uv[:, :, 1]
        ue = jnp.dot(u, spe, preferred_element_type=jnp.float32)
        ve = jnp.dot(v, dpe, preferred_element_type=jnp.float32)
        e = ue + ve + ewp[:, :, l]
        e = jnp.where(e > 0, e, 0.2 * e)
        em = e[:, None, :] + mb[None, :, :]
        m = jnp.max(em, axis=2)
        me = jnp.dot(m, dpe, preferred_element_type=jnp.float32)
        ex = jnp.exp(e - me)
        s = jnp.dot(ex, dep, preferred_element_type=jnp.float32)
        se = jnp.dot(s, dpe, preferred_element_type=jnp.float32)
        alpha = ex / (se + 1e-9)
        aal = dpe[None, :, :] * alpha[:, None, :]
        t = jax.lax.dot_general(aal, spe, (((2,), (1,)), ((), ())),
                                preferred_element_type=jnp.float32)
        agg = jax.lax.dot_general(t, z, (((2,), (1,)), ((0,), (0,))),
                                  preferred_element_type=jnp.float32)
        h = jnp.where(agg > 0, agg, jnp.exp(agg) - 1.0) + h
        acc[l, :] += jnp.sum(h, axis=(0, 1))

    @pl.when(i == pl.num_programs(0) - 1)
    def _():
        mesh = acc[...] * (1.0 / (NP * P))
        o = jnp.zeros((1, OUT), jnp.float32)
        for l in range(3):
            o = o + jnp.dot(mesh[l:l + 1, :], wcls[l * DH:(l + 1) * DH, :],
                            preferred_element_type=jnp.float32)
        out[...] = o


@jax.jit
def kernel(x, edge_index, edge_w, W0, W1, W2, asrc0, asrc1, asrc2,
           adst0, adst1, adst2, we0, we1, we2, Wcls):
    src = edge_index[0].astype(jnp.int32)
    dst = edge_index[1].astype(jnp.int32)
    nodes = jnp.arange(P, dtype=jnp.int32)
    srcpe = (src[None, :] == nodes[:, None]).astype(jnp.float32)
    dstpe = (dst[None, :] == nodes[:, None]).astype(jnp.float32)
    dstep = dstpe.T
    mbias = (dstpe - 1.0) * 1e30

    am0 = jnp.stack([asrc0, adst0], axis=1)
    am1 = jnp.stack([asrc1, adst1], axis=1)
    am2 = jnp.stack([asrc2, adst2], axis=1)
    wem = jnp.stack([we0, we1, we2], axis=1)

    full = lambda a: pl.BlockSpec(a.shape, lambda i: (0,) * a.ndim)
    grid_spec = pltpu.PrefetchScalarGridSpec(
        num_scalar_prefetch=0,
        grid=(G,),
        in_specs=[
            pl.BlockSpec((B, P, DF), lambda i: (i, 0, 0)),
            pl.BlockSpec((B, E, DW), lambda i: (i, 0, 0)),
            full(W0), full(W1), full(W2),
            full(am0), full(am1), full(am2), full(wem),
            full(srcpe), full(dstpe), full(dstep), full(mbias), full(Wcls),
        ],
        out_specs=pl.BlockSpec((1, OUT), lambda i: (0, 0)),
        scratch_shapes=[pltpu.VMEM((3, DH), jnp.float32)],
    )
    return pl.pallas_call(
        _gat_body,
        grid_spec=grid_spec,
        out_shape=jax.ShapeDtypeStruct((1, OUT), jnp.float32),
        compiler_params=pltpu.CompilerParams(
            dimension_semantics=("arbitrary",),
        ),
    )(x, edge_w, W0, W1, W2, am0, am1, am2, wem,
      srcpe, dstpe, dstep, mbias, Wcls)


# speedup vs baseline: 5.0979x; 1.7776x over previous
import jax
import jax.numpy as jnp
from jax.experimental import pallas as pl
from jax.experimental.pallas import tpu as pltpu

NP = 4000
P = 50
E = 200
DF = 64
DH = 64
DW = 16
OUT = 16
B = 80
G = NP // B


def _gat_body(xb, ewb, wa0, wa1, wa2, ewk,
              srcpe, dstpe, dstep, scat, wcls, out, acc):
    i = pl.program_id(0)

    @pl.when(i == 0)
    def _():
        acc[...] = jnp.zeros_like(acc)

    spe = srcpe[...]
    dpe = dstpe[...]
    dep = dstep[...]

    ewp = jax.lax.dot_general(ewb[...], ewk[...], (((1,), (0,)), ((), ())),
                              preferred_element_type=jnp.float32)

    h = xb[...]
    for l, war in enumerate((wa0, wa1, wa2)):
        za = jax.lax.dot_general(h, war[...], (((2,), (0,)), ((), ())),
                                 preferred_element_type=jnp.float32)
        z = za[:, :, 0:DH]
        u = za[:, :, DH]
        v = za[:, :, DH + 1]
        ue = jnp.dot(u, spe, preferred_element_type=jnp.float32)
        ve = jnp.dot(v, dpe, preferred_element_type=jnp.float32)
        e = ue + ve + ewp[:, l * E:(l + 1) * E]
        e = jnp.where(e > 0, e, 0.2 * e)
        c = jnp.max(e, axis=1, keepdims=True)
        ex1 = jnp.exp(e - c)
        s1 = jnp.dot(ex1, dep, preferred_element_type=jnp.float32)
        s1e = jnp.dot(s1, dpe, preferred_element_type=jnp.float32)
        alpha = ex1 / (s1e + 1e-30)
        tf = jnp.dot(alpha, scat[...], preferred_element_type=jnp.float32)
        t = tf.reshape(B, P, P)
        agg = jax.lax.dot_general(t, z, (((2,), (1,)), ((0,), (0,))),
                                  preferred_element_type=jnp.float32)
        h = jnp.where(agg > 0, agg, jnp.exp(agg) - 1.0) + h
        acc[l, :] += jnp.sum(h, axis=(0, 1))

    @pl.when(i == pl.num_programs(0) - 1)
    def _():
        mesh = acc[...] * (1.0 / (NP * P))
        o = jnp.zeros((1, OUT), jnp.float32)
        for l in range(3):
            o = o + jnp.dot(mesh[l:l + 1, :], wcls[l * DH:(l + 1) * DH, :],
                            preferred_element_type=jnp.float32)
        out[...] = o


@jax.jit
def kernel(x, edge_index, edge_w, W0, W1, W2, asrc0, asrc1, asrc2,
           adst0, adst1, adst2, we0, we1, we2, Wcls):
    src = edge_index[0].astype(jnp.int32)
    dst = edge_index[1].astype(jnp.int32)
    nodes = jnp.arange(P, dtype=jnp.int32)
    srcpe = (src[None, :] == nodes[:, None]).astype(jnp.float32)
    dstpe = (dst[None, :] == nodes[:, None]).astype(jnp.float32)
    dstep = dstpe.T
    scat = (dst * P + src)[:, None] == jnp.arange(P * P, dtype=jnp.int32)[None, :]
    scat = scat.astype(jnp.float32)

    wa0 = jnp.concatenate([W0, (W0 @ asrc0)[:, None], (W0 @ adst0)[:, None]], axis=1)
    wa1 = jnp.concatenate([W1, (W1 @ asrc1)[:, None], (W1 @ adst1)[:, None]], axis=1)
    wa2 = jnp.concatenate([W2, (W2 @ asrc2)[:, None], (W2 @ adst2)[:, None]], axis=1)

    wem = jnp.stack([we0, we1, we2], axis=1)
    eye = jnp.eye(E, dtype=jnp.float32)
    ewk = jnp.einsum('ef,wl->ewlf', eye, wem).reshape(E * DW, 3 * E)

    ew2 = edge_w.reshape(NP, E * DW)

    full = lambda a: pl.BlockSpec(a.shape, lambda i: (0,) * a.ndim)
    grid_spec = pltpu.PrefetchScalarGridSpec(
        num_scalar_prefetch=0,
        grid=(G,),
        in_specs=[
            pl.BlockSpec((B, P, DF), lambda i: (i, 0, 0)),
            pl.BlockSpec((B, E * DW), lambda i: (i, 0)),
            full(wa0), full(wa1), full(wa2), full(ewk),
            full(srcpe), full(dstpe), full(dstep), full(scat), full(Wcls),
        ],
        out_specs=pl.BlockSpec((1, OUT), lambda i: (0, 0)),
        scratch_shapes=[pltpu.VMEM((3, DH), jnp.float32)],
    )
    return pl.pallas_call(
        _gat_body,
        grid_spec=grid_spec,
        out_shape=jax.ShapeDtypeStruct((1, OUT), jnp.float32),
        compiler_params=pltpu.CompilerParams(
            dimension_semantics=("arbitrary",),
        ),
    )(x, ew2, wa0, wa1, wa2, ewk,
      srcpe, dstpe, dstep, scat, Wcls)


# speedup vs baseline: 5.7109x; 1.1202x over previous
import jax
import jax.numpy as jnp
from jax.experimental import pallas as pl
from jax.experimental.pallas import tpu as pltpu

NP = 4000
P = 50
E = 200
DF = 64
DH = 64
DW = 16
OUT = 16
B = 200
G = NP // B


def _gat_body(xb, ewb, wa0, wa1, wa2, ewk,
              srcpe, dstpe, dstep, scat, wcls, out, acc):
    i = pl.program_id(0)

    @pl.when(i == 0)
    def _():
        acc[...] = jnp.zeros_like(acc)

    spe = srcpe[...]
    dpe = dstpe[...]
    dep = dstep[...]

    ewp = jax.lax.dot_general(ewb[...], ewk[...], (((1,), (0,)), ((), ())),
                              preferred_element_type=jnp.float32)

    h = xb[...]
    for l, war in enumerate((wa0, wa1, wa2)):
        za = jax.lax.dot_general(h.astype(jnp.bfloat16), war[...],
                                 (((2,), (0,)), ((), ())),
                                 preferred_element_type=jnp.float32)
        z = za[:, :, 0:DH]
        u = za[:, :, DH]
        v = za[:, :, DH + 1]
        ue = jnp.dot(u, spe, preferred_element_type=jnp.float32)
        ve = jnp.dot(v, dpe, preferred_element_type=jnp.float32)
        e = ue + ve + ewp[:, l * E:(l + 1) * E]
        e = jnp.where(e > 0, e, 0.2 * e)
        c = jnp.max(e, axis=1, keepdims=True)
        ex1 = jnp.exp(e - c)
        s1 = jnp.dot(ex1, dep, preferred_element_type=jnp.float32)
        s1e = jnp.dot(s1, dpe, preferred_element_type=jnp.float32)
        alpha = ex1 / (s1e + 1e-30)
        tf = jnp.dot(alpha, scat[...], preferred_element_type=jnp.float32)
        t = tf.reshape(B, P, P)
        agg = jax.lax.dot_general(t, z, (((2,), (1,)), ((0,), (0,))),
                                  preferred_element_type=jnp.float32)
        h = jnp.where(agg > 0, agg, jnp.exp(agg) - 1.0) + h
        acc[l, :] += jnp.sum(h, axis=(0, 1))

    @pl.when(i == pl.num_programs(0) - 1)
    def _():
        mesh = acc[...] * (1.0 / (NP * P))
        o = jnp.zeros((1, OUT), jnp.float32)
        for l in range(3):
            o = o + jnp.dot(mesh[l:l + 1, :], wcls[l * DH:(l + 1) * DH, :],
                            preferred_element_type=jnp.float32)
        out[...] = o


@jax.jit
def kernel(x, edge_index, edge_w, W0, W1, W2, asrc0, asrc1, asrc2,
           adst0, adst1, adst2, we0, we1, we2, Wcls):
    src = edge_index[0].astype(jnp.int32)
    dst = edge_index[1].astype(jnp.int32)
    nodes = jnp.arange(P, dtype=jnp.int32)
    srcpe = (src[None, :] == nodes[:, None]).astype(jnp.float32)
    dstpe = (dst[None, :] == nodes[:, None]).astype(jnp.float32)
    dstep = dstpe.T
    scat = (dst * P + src)[:, None] == jnp.arange(P * P, dtype=jnp.int32)[None, :]
    scat = scat.astype(jnp.float32)

    wa0 = jnp.concatenate([W0, (W0 @ asrc0)[:, None], (W0 @ adst0)[:, None]], axis=1)
    wa1 = jnp.concatenate([W1, (W1 @ asrc1)[:, None], (W1 @ adst1)[:, None]], axis=1)
    wa2 = jnp.concatenate([W2, (W2 @ asrc2)[:, None], (W2 @ adst2)[:, None]], axis=1)
    wa0, wa1, wa2 = (w.astype(jnp.bfloat16) for w in (wa0, wa1, wa2))

    wem = jnp.stack([we0, we1, we2], axis=1)
    eye = jnp.eye(E, dtype=jnp.float32)
    ewk = jnp.einsum('ef,wl->ewlf', eye, wem).reshape(E * DW, 3 * E)

    ew2 = edge_w.reshape(NP, E * DW)

    full = lambda a: pl.BlockSpec(a.shape, lambda i: (0,) * a.ndim)
    grid_spec = pltpu.PrefetchScalarGridSpec(
        num_scalar_prefetch=0,
        grid=(G,),
        in_specs=[
            pl.BlockSpec((B, P, DF), lambda i: (i, 0, 0)),
            pl.BlockSpec((B, E * DW), lambda i: (i, 0)),
            full(wa0), full(wa1), full(wa2), full(ewk),
            full(srcpe), full(dstpe), full(dstep), full(scat), full(Wcls),
        ],
        out_specs=pl.BlockSpec((1, OUT), lambda i: (0, 0)),
        scratch_shapes=[pltpu.VMEM((3, DH), jnp.float32)],
    )
    return pl.pallas_call(
        _gat_body,
        grid_spec=grid_spec,
        out_shape=jax.ShapeDtypeStruct((1, OUT), jnp.float32),
        compiler_params=pltpu.CompilerParams(
            dimension_semantics=("arbitrary",),
        ),
    )(x, ew2, wa0, wa1, wa2, ewk,
      srcpe, dstpe, dstep, scat, Wcls)


# speedup vs baseline: 5.9432x; 1.0407x over previous
import jax
import jax.numpy as jnp
from jax.experimental import pallas as pl
from jax.experimental.pallas import tpu as pltpu

NP = 4000
P = 50
E = 200
DF = 64
DH = 64
DW = 16
OUT = 16
B = 200
G = NP // B


def _gat_body(xb, ewb, ei, w0, w1, w2, a0, a1, a2, wes, wcls, out, acc):
    i = pl.program_id(0)

    @pl.when(i == 0)
    def _():
        acc[...] = jnp.zeros_like(acc)

    eii = ei[...]
    srcr = eii[0:1, :]
    dstr = eii[1:2, :]
    iota_pe = jax.lax.broadcasted_iota(jnp.int32, (P, E), 0)
    spe = (iota_pe == srcr).astype(jnp.float32)
    dpe = (iota_pe == dstr).astype(jnp.float32)
    dsc = jnp.transpose(dstr * P + srcr)
    iota_epp = jax.lax.broadcasted_iota(jnp.int32, (E, P * P), 1)
    scat = (iota_epp == dsc).astype(jnp.float32)
    ir = jax.lax.broadcasted_iota(jnp.int32, (E * DW, E), 0) // DW
    ic = jax.lax.broadcasted_iota(jnp.int32, (E * DW, E), 1)
    emask = (ir == ic).astype(jnp.float32)

    ew = ewb[...]
    ewps = []
    for l in range(3):
        wecol = jnp.transpose(wes[...][l:l + 1, :])
        wecol = jnp.broadcast_to(wecol[None, :, :], (E, DW, 1)).reshape(E * DW, 1)
        ewps.append(jnp.dot(ew, emask * wecol, preferred_element_type=jnp.float32))

    h = xb[...]
    for l, (wr, ar) in enumerate(((w0, a0), (w1, a1), (w2, a2))):
        wuv = jax.lax.dot_general(wr[...], ar[...], (((1,), (1,)), ((), ())),
                                  preferred_element_type=jnp.float32)
        war = jnp.concatenate([wr[...], wuv], axis=1)
        za = jax.lax.dot_general(h, war, (((2,), (0,)), ((), ())),
                                 preferred_element_type=jnp.float32)
        z = za[:, :, 0:DH]
        u = za[:, :, DH]
        v = za[:, :, DH + 1]
        ue = jnp.dot(u, spe, preferred_element_type=jnp.float32)
        ve = jnp.dot(v, dpe, preferred_element_type=jnp.float32)
        e = ue + ve + ewps[l]
        e = jnp.where(e > 0, e, 0.2 * e)
        c = jnp.max(e, axis=1, keepdims=True)
        ex1 = jnp.exp(e - c)
        s1 = jax.lax.dot_general(ex1, dpe, (((1,), (1,)), ((), ())),
                                 preferred_element_type=jnp.float32)
        s1e = jnp.dot(s1, dpe, preferred_element_type=jnp.float32)
        alpha = ex1 / (s1e + 1e-30)
        tf = jnp.dot(alpha, scat, preferred_element_type=jnp.float32)
        t = tf.reshape(B, P, P)
        agg = jax.lax.dot_general(t, z, (((2,), (1,)), ((0,), (0,))),
                                  preferred_element_type=jnp.float32)
        h = jnp.where(agg > 0, agg, jnp.exp(agg) - 1.0) + h
        acc[l, :] += jnp.sum(h, axis=(0, 1))

    @pl.when(i == pl.num_programs(0) - 1)
    def _():
        mesh = acc[...] * (1.0 / (NP * P))
        o = jnp.zeros((1, OUT), jnp.float32)
        for l in range(3):
            o = o + jnp.dot(mesh[l:l + 1, :], wcls[l * DH:(l + 1) * DH, :],
                            preferred_element_type=jnp.float32)
        out[...] = o


@jax.jit
def kernel(x, edge_index, edge_w, W0, W1, W2, asrc0, asrc1, asrc2,
           adst0, adst1, adst2, we0, we1, we2, Wcls):
    ei = edge_index.astype(jnp.int32)
    ew2 = edge_w.reshape(NP, E * DW)
    a0 = jnp.stack([asrc0, adst0], axis=0)
    a1 = jnp.stack([asrc1, adst1], axis=0)
    a2 = jnp.stack([asrc2, adst2], axis=0)
    wes = jnp.stack([we0, we1, we2], axis=0)

    full = lambda a: pl.BlockSpec(a.shape, lambda i: (0,) * a.ndim)
    grid_spec = pltpu.PrefetchScalarGridSpec(
        num_scalar_prefetch=0,
        grid=(G,),
        in_specs=[
            pl.BlockSpec((B, P, DF), lambda i: (i, 0, 0)),
            pl.BlockSpec((B, E * DW), lambda i: (i, 0)),
            full(ei), full(W0), full(W1), full(W2),
            full(a0), full(a1), full(a2), full(wes), full(Wcls),
        ],
        out_specs=pl.BlockSpec((1, OUT), lambda i: (0, 0)),
        scratch_shapes=[pltpu.VMEM((3, DH), jnp.float32)],
    )
    return pl.pallas_call(
        _gat_body,
        grid_spec=grid_spec,
        out_shape=jax.ShapeDtypeStruct((1, OUT), jnp.float32),
        compiler_params=pltpu.CompilerParams(
            dimension_semantics=("arbitrary",),
        ),
    )(x, ew2, ei, W0, W1, W2, a0, a1, a2, wes, Wcls)


# speedup vs baseline: 5.9672x; 1.0040x over previous
import jax
import jax.numpy as jnp
from jax.experimental import pallas as pl
from jax.experimental.pallas import tpu as pltpu

NP = 4000
P = 50
E = 200
DF = 64
DH = 64
DW = 16
OUT = 16
B = 200
G = NP // B


def _gat_body(xb, ewb, ei, w0, w1, w2, as0, as1, as2, ad0, ad1, ad2,
              we0r, we1r, we2r, wcls, out, acc):
    i = pl.program_id(0)

    @pl.when(i == 0)
    def _():
        acc[...] = jnp.zeros_like(acc)

    eii = ei[...]
    srcr = eii[0:1, :]
    dstr = eii[1:2, :]
    iota_pe = jax.lax.broadcasted_iota(jnp.int32, (P, E), 0)
    spe = (iota_pe == srcr).astype(jnp.float32)
    dpe = (iota_pe == dstr).astype(jnp.float32)
    dsc = jnp.transpose(dstr * P + srcr)
    iota_epp = jax.lax.broadcasted_iota(jnp.int32, (E, P * P), 1)
    scat = (iota_epp == dsc).astype(jnp.float32)
    ir = jax.lax.broadcasted_iota(jnp.int32, (E * DW, E), 0) // DW
    ic = jax.lax.broadcasted_iota(jnp.int32, (E * DW, E), 1)
    emask = (ir == ic).astype(jnp.float32)

    ew = ewb[...]
    ewps = []
    for wer in (we0r, we1r, we2r):
        wecol = jnp.transpose(wer[...])
        wecol = jnp.broadcast_to(wecol[None, :, :], (E, DW, 1)).reshape(E * DW, 1)
        ewps.append(jnp.dot(ew, emask * wecol, preferred_element_type=jnp.float32))

    h = xb[...]
    for l, (wr, asr, adr) in enumerate(((w0, as0, ad0), (w1, as1, ad1),
                                        (w2, as2, ad2))):
        ar = jnp.concatenate([asr[...], adr[...]], axis=0)
        wuv = jax.lax.dot_general(wr[...], ar, (((1,), (1,)), ((), ())),
                                  preferred_element_type=jnp.float32)
        war = jnp.concatenate([wr[...], wuv], axis=1)
        za = jax.lax.dot_general(h, war, (((2,), (0,)), ((), ())),
                                 preferred_element_type=jnp.float32)
        z = za[:, :, 0:DH]
        u = za[:, :, DH]
        v = za[:, :, DH + 1]
        ue = jnp.dot(u, spe, preferred_element_type=jnp.float32)
        ve = jnp.dot(v, dpe, preferred_element_type=jnp.float32)
        e = ue + ve + ewps[l]
        e = jnp.where(e > 0, e, 0.2 * e)
        c = jnp.max(e, axis=1, keepdims=True)
        ex1 = jnp.exp(e - c)
        s1 = jax.lax.dot_general(ex1, dpe, (((1,), (1,)), ((), ())),
                                 preferred_element_type=jnp.float32)
        s1e = jnp.dot(s1, dpe, preferred_element_type=jnp.float32)
        alpha = ex1 / (s1e + 1e-30)
        tf = jnp.dot(alpha, scat, preferred_element_type=jnp.float32)
        t = tf.reshape(B, P, P)
        agg = jax.lax.dot_general(t, z, (((2,), (1,)), ((0,), (0,))),
                                  preferred_element_type=jnp.float32)
        h = jnp.where(agg > 0, agg, jnp.exp(agg) - 1.0) + h
        acc[l, :] += jnp.sum(h, axis=(0, 1))

    @pl.when(i == pl.num_programs(0) - 1)
    def _():
        mesh = acc[...] * (1.0 / (NP * P))
        o = jnp.zeros((1, OUT), jnp.float32)
        for l in range(3):
            o = o + jnp.dot(mesh[l:l + 1, :], wcls[l * DH:(l + 1) * DH, :],
                            preferred_element_type=jnp.float32)
        out[...] = o


@jax.jit
def kernel(x, edge_index, edge_w, W0, W1, W2, asrc0, asrc1, asrc2,
           adst0, adst1, adst2, we0, we1, we2, Wcls):
    ei = edge_index.astype(jnp.int32)
    ew2 = edge_w.reshape(NP, E * DW)
    as0, as1, as2 = (a.reshape(1, DH) for a in (asrc0, asrc1, asrc2))
    ad0, ad1, ad2 = (a.reshape(1, DH) for a in (adst0, adst1, adst2))
    we0r, we1r, we2r = (w.reshape(1, DW) for w in (we0, we1, we2))

    full = lambda a: pl.BlockSpec(a.shape, lambda i: (0,) * a.ndim)
    grid_spec = pltpu.PrefetchScalarGridSpec(
        num_scalar_prefetch=0,
        grid=(G,),
        in_specs=[
            pl.BlockSpec((B, P, DF), lambda i: (i, 0, 0)),
            pl.BlockSpec((B, E * DW), lambda i: (i, 0)),
            full(ei), full(W0), full(W1), full(W2),
            full(as0), full(as1), full(as2),
            full(ad0), full(ad1), full(ad2),
            full(we0r), full(we1r), full(we2r), full(Wcls),
        ],
        out_specs=pl.BlockSpec((1, OUT), lambda i: (0, 0)),
        scratch_shapes=[pltpu.VMEM((3, DH), jnp.float32)],
    )
    return pl.pallas_call(
        _gat_body,
        grid_spec=grid_spec,
        out_shape=jax.ShapeDtypeStruct((1, OUT), jnp.float32),
        compiler_params=pltpu.CompilerParams(
            dimension_semantics=("arbitrary",),
        ),
    )(x, ew2, ei, W0, W1, W2, as0, as1, as2, ad0, ad1, ad2,
      we0r, we1r, we2r, Wcls)
